# baseline (speedup 1.0000x reference)
import functools

import jax
import jax.numpy as jnp
from jax import lax
from jax.experimental import pallas as pl
from jax.experimental.pallas import tpu as pltpu
from jax.experimental.pallas import tpu_sc as plsc

NW = 32
CHUNK = 128


def _embed_kernel(n_total, d, nchunk):
    per_w = n_total // NW
    mesh = plsc.VectorSubcoreMesh(core_axis_name="c", subcore_axis_name="s")

    @functools.partial(
        pl.kernel,
        mesh=mesh,
        out_type=jax.ShapeDtypeStruct((n_total, d), jnp.float32),
        scratch_types=[
            pltpu.VMEM((nchunk, CHUNK), jnp.int32),
            pltpu.VMEM((CHUNK, d), jnp.float32),
            pltpu.SemaphoreType.DMA,
        ],
        compiler_params=pltpu.CompilerParams(use_tc_tiling_on_sc=False),
    )
    def k(idx_hbm, table_hbm, out_hbm, idx_v, rows_v, gsem):
        wid = lax.axis_index("s") * 2 + lax.axis_index("c")
        base = wid * per_w
        pltpu.sync_copy(idx_hbm.at[wid], idx_v)

        def body(j, carry):
            pltpu.async_copy(table_hbm.at[idx_v.at[j]], rows_v, gsem).wait()
            pltpu.sync_copy(rows_v, out_hbm.at[pl.ds(base + j * CHUNK, CHUNK)])
            return carry

        lax.fori_loop(0, nchunk, body, 0)

    return k


def kernel(indices, table):
    b, l = indices.shape
    v, d = table.shape
    n_total = b * l
    per_w = n_total // NW
    nchunk = per_w // CHUNK
    idx = indices.reshape(NW, nchunk, CHUNK).astype(jnp.int32)
    out = _embed_kernel(n_total, d, nchunk)(idx, table)
    return out.reshape(b, l, d)


# speedup vs baseline: 1.1169x; 1.1169x over previous
import functools

import jax
import jax.numpy as jnp
from jax import lax
from jax.experimental import pallas as pl
from jax.experimental.pallas import tpu as pltpu
from jax.experimental.pallas import tpu_sc as plsc

NW = 32
CHUNK = 256
NBUF = 5


def _embed_kernel(n_total, d, ng):
    per_w = n_total // NW
    mesh = plsc.VectorSubcoreMesh(core_axis_name="c", subcore_axis_name="s")

    @functools.partial(
        pl.kernel,
        mesh=mesh,
        out_type=jax.ShapeDtypeStruct((n_total, d), jnp.float32),
        scratch_types=[
            pltpu.VMEM((ng, CHUNK), jnp.int32),
            pltpu.VMEM((NBUF, CHUNK, d), jnp.float32),
            [pltpu.SemaphoreType.DMA] * NBUF,
            [pltpu.SemaphoreType.DMA] * NBUF,
        ],
        compiler_params=pltpu.CompilerParams(use_tc_tiling_on_sc=False),
    )
    def k(idx_hbm, table_hbm, out_hbm, idx_v, rows_v, gsems, wsems):
        wid = lax.axis_index("s") * 2 + lax.axis_index("c")
        base = wid * per_w
        pltpu.sync_copy(idx_hbm.at[wid], idx_v)

        def gather_args(j, b):
            return table_hbm.at[idx_v.at[j]], rows_v.at[b], gsems[b]

        def write_args(j, b):
            return rows_v.at[b], out_hbm.at[pl.ds(base + j * CHUNK, CHUNK)], wsems[b]

        for b in range(min(NBUF - 2, ng)):
            pltpu.async_copy(*gather_args(b, b))

        def slot(j, b):
            @pl.when(j >= 2)
            def _():
                pltpu.make_async_copy(*write_args(j - 2, (b - 2) % NBUF)).wait()

            @pl.when(j + NBUF - 2 < ng)
            def _():
                pltpu.async_copy(*gather_args(j + NBUF - 2, (b + NBUF - 2) % NBUF))

            pltpu.make_async_copy(*gather_args(j, b)).wait()
            pltpu.async_copy(*write_args(j, b))

        def body(g, carry):
            for b in range(NBUF):
                slot(g + b, b)
            return carry

        lax.fori_loop(0, ng // NBUF, lambda i, c: body(i * NBUF, c), 0)

        for j in (ng - 2, ng - 1):
            pltpu.make_async_copy(*write_args(j, j % NBUF)).wait()

    return k


def kernel(indices, table):
    b, l = indices.shape
    v, d = table.shape
    n_total = b * l
    per_w = n_total // NW
    ng = per_w // CHUNK
    idx = indices.reshape(NW, ng, CHUNK).astype(jnp.int32)
    out = _embed_kernel(n_total, d, ng)(idx, table)
    return out.reshape(b, l, d)
